# Initial kernel scaffold; baseline (speedup 1.0000x reference)
import jax
import jax.numpy as jnp
from jax.experimental import pallas as pl


def kernel(x):
    raise NotImplementedError("write your pallas kernel here")


# speedup vs baseline: 32.5982x; 32.5982x over previous
import jax
import jax.numpy as jnp
from jax import lax
from jax.experimental import pallas as pl
from jax.experimental.pallas import tpu as pltpu
from jax.experimental.pallas import tpu_sc as plsc

B = 64
N = 262144
NW = 32
ROWS_PER_W = B // NW
CHUNK = 8192
NCHUNK = N // CHUNK
NV = CHUNK // 16

POS_LO = 2621.43
POS_UP = 259521.58
T_LO = 2621
T_UP = 259521

_INT_MIN_PY = -2147483648


def _sc_body(x_hbm, out_hbm, hist, coarse, buf0, buf1, outv, sem0, sem1):
    _INT_MIN = jnp.int32(_INT_MIN_PY)
    cid = lax.axis_index("c")
    sid = lax.axis_index("s")
    wid = sid * 2 + cid
    bufs = (buf0, buf1)
    sems = (sem0, sem1)
    iota = lax.iota(jnp.int32, 16)
    zeros16 = jnp.zeros((16,), jnp.int32)
    ones16 = jnp.ones((16,), jnp.int32)

    for rr in range(ROWS_PER_W):
        row = wid * ROWS_PER_W + rr

        def _zero_hist(i, _):
            hist[pl.ds(i * 16, 16)] = zeros16
            return 0

        lax.fori_loop(0, 4096, _zero_hist, 0)

        def _zero_coarse(i, _):
            coarse[pl.ds(i * 16, 16)] = zeros16
            return 0

        lax.fori_loop(0, 256, _zero_coarse, 0)

        pltpu.make_async_copy(
            x_hbm.at[row, pl.ds(0, CHUNK)], bufs[0], sems[0]).start()
        for c in range(NCHUNK):
            buf = bufs[c % 2]
            pltpu.make_async_copy(
                x_hbm.at[row, pl.ds(c * CHUNK, CHUNK)], buf, sems[c % 2]).wait()
            if c + 1 < NCHUNK:
                pltpu.make_async_copy(
                    x_hbm.at[row, pl.ds((c + 1) * CHUNK, CHUNK)],
                    bufs[(c + 1) % 2], sems[(c + 1) % 2]).start()

            def _hist_body(j, _):
                v = buf[pl.ds(j * 16, 16)]
                k32 = plsc.bitcast(v, jnp.int32)
                m = lax.shift_right_arithmetic(k32, 31)
                key = lax.bitwise_xor(k32, lax.bitwise_or(m, _INT_MIN))
                bkt = lax.shift_right_logical(key, 16)
                cbkt = lax.shift_right_logical(key, 20)
                plsc.addupdate_scatter(hist, [bkt], ones16)
                plsc.addupdate_scatter(coarse, [cbkt], ones16)
                return 0

            lax.fori_loop(0, NV, _hist_body, 0)

        def _cscan(g, carry):
            cum, cbL, cumbefL, cbU, cumbefU = carry
            vec = coarse[pl.ds(g * 16, 16)]
            s = plsc.cumsum(vec)
            tot = jnp.sum(vec)
            sbef = s - vec

            def locate(t, cb, cumbef):
                cond = (cum <= t) & (t < cum + tot)
                lane = jnp.sum((cum + s <= t).astype(jnp.int32))
                sel = jnp.sum(jnp.where(iota == lane, sbef, 0))
                cb2 = lax.select(cond, g * 16 + lane, cb)
                cumbef2 = lax.select(cond, cum + sel, cumbef)
                return cb2, cumbef2

            cbL, cumbefL = locate(jnp.int32(T_LO), cbL, cumbefL)
            cbU, cumbefU = locate(jnp.int32(T_UP), cbU, cumbefU)
            return cum + tot, cbL, cumbefL, cbU, cumbefU

        z = jnp.int32(0)
        _, cbL, cumbefL, cbU, cumbefU = lax.fori_loop(
            0, 256, _cscan, (z, z, z, z, z))

        def _fine(cb, cumbef, t):
            vec = hist[pl.ds(cb * 16, 16)]
            s = plsc.cumsum(vec)
            r = jnp.int32(t) - cumbef
            lane = jnp.sum((s <= r).astype(jnp.int32))
            cnt = jnp.sum(jnp.where(iota == lane, vec, 0))
            cbef = cumbef + jnp.sum(jnp.where(iota == lane, s - vec, 0))
            fbin = cb * 16 + lane
            klo = lax.shift_left(fbin, 16)
            khi = lax.shift_left(fbin + 1, 16)
            kv = jnp.where(iota == 0, klo, khi)
            bits = jnp.where(kv < 0,
                             lax.bitwise_xor(kv, _INT_MIN),
                             lax.bitwise_not(kv))
            fv = plsc.bitcast(bits, jnp.float32)
            neg_big = jnp.float32(-3.4e38)
            vlo = jnp.max(jnp.where(iota == 0, fv, neg_big))
            vhi = jnp.max(jnp.where(iota == 1, fv, neg_big))
            return vlo, vhi, cbef.astype(jnp.float32), cnt.astype(jnp.float32)

        vloL, vhiL, cbefL, cntL = _fine(cbL, cumbefL, T_LO)
        vloU, vhiU, cbefU, cntU = _fine(cbU, cumbefU, T_UP)
        fz = jnp.float32(0.0)
        resv = fz
        for lane_ix, val in ((0, vloL), (1, vhiL), (2, cbefL), (3, cntL),
                             (4, vloU), (5, vhiU), (6, cbefU), (7, cntU)):
            resv = jnp.where(iota == lane_ix, val, resv)
        outv[...] = resv
        pltpu.sync_copy(outv, out_hbm.at[row])


_sc_quantile = pl.kernel(
    _sc_body,
    out_type=jax.ShapeDtypeStruct((B, 16), jnp.float32),
    mesh=plsc.VectorSubcoreMesh(core_axis_name="c", subcore_axis_name="s"),
    compiler_params=pltpu.CompilerParams(needs_layout_passes=False),
    scratch_types=[
        pltpu.VMEM((65536,), jnp.int32),
        pltpu.VMEM((4096,), jnp.int32),
        pltpu.VMEM((CHUNK,), jnp.float32),
        pltpu.VMEM((CHUNK,), jnp.float32),
        pltpu.VMEM((16,), jnp.float32),
        pltpu.SemaphoreType.DMA,
        pltpu.SemaphoreType.DMA,
    ],
)


def _norm_body(b_ref, x_ref, o_ref):
    def interp(vlo, vhi, cbef, cnt, pos):
        return vlo + (vhi - vlo) * ((jnp.float32(pos) - cbef
                                     + jnp.float32(0.5)) / cnt)

    lo = interp(b_ref[0, 0, 0], b_ref[0, 0, 1], b_ref[0, 0, 2],
                b_ref[0, 0, 3], POS_LO)
    up = interp(b_ref[0, 0, 4], b_ref[0, 0, 5], b_ref[0, 0, 6],
                b_ref[0, 0, 7], POS_UP)
    xv = x_ref[...]
    o_ref[...] = (jnp.maximum(jnp.minimum(xv, up), lo) - lo) / (up - lo)


_tc_normalize = pl.pallas_call(
    _norm_body,
    grid=(B,),
    in_specs=[
        pl.BlockSpec((1, 1, 16), lambda i: (i, 0, 0)),
        pl.BlockSpec((1, 1, 512, 512), lambda i: (i, 0, 0, 0)),
    ],
    out_specs=pl.BlockSpec((1, 1, 512, 512), lambda i: (i, 0, 0, 0)),
    out_shape=jax.ShapeDtypeStruct((B, 1, 512, 512), jnp.float32),
)


def kernel(x):
    xf = x.reshape(B, N)
    bounds = _sc_quantile(xf)
    return _tc_normalize(bounds.reshape(B, 1, 16), x)


# speedup vs baseline: 38.9303x; 1.1942x over previous
import jax
import jax.numpy as jnp
from jax import lax
from jax.experimental import pallas as pl
from jax.experimental.pallas import tpu as pltpu
from jax.experimental.pallas import tpu_sc as plsc

B = 64
N = 262144
ROWS = 512
NW = 32
ROWS_PER_W = B // NW
CROWS = 16
CHUNK = CROWS * 512
NCHUNK = N // CHUNK
NV = CHUNK // 16
UNROLL = 8

POS_LO = 2621.43
POS_UP = 259521.58
T_LO = 2621
T_UP = 259521

_INT_MIN_PY = -2147483648


def _sc_body(x_hbm, out_hbm, hist, coarse, buf0, buf1, outv, sem0, sem1):
    _INT_MIN = jnp.int32(_INT_MIN_PY)
    cid = lax.axis_index("c")
    sid = lax.axis_index("s")
    wid = sid * 2 + cid
    iota = lax.iota(jnp.int32, 16)
    zeros16 = jnp.zeros((16,), jnp.int32)
    ones16 = jnp.ones((16,), jnp.int32)

    def start_chunk(row, c, buf, sem):
        pltpu.make_async_copy(
            x_hbm.at[row, pl.ds(c * CROWS, CROWS)], buf, sem).start()

    def wait_chunk(row, c, buf, sem):
        pltpu.make_async_copy(
            x_hbm.at[row, pl.ds(c * CROWS, CROWS)], buf, sem).wait()

    def process(buf):
        def _hist_body(j, _):
            j0 = j * UNROLL
            for u in range(UNROLL):
                jj = j0 + u
                ri = lax.shift_right_logical(jj, 5)
                col = lax.shift_left(jnp.bitwise_and(jj, 31), 4)
                v = buf[ri, pl.ds(col, 16)]
                k32 = plsc.bitcast(v, jnp.int32)
                m = lax.shift_right_arithmetic(k32, 31)
                key = lax.bitwise_xor(k32, lax.bitwise_or(m, _INT_MIN))
                bkt = lax.shift_right_logical(key, 16)
                cbkt = lax.shift_right_logical(bkt, 4)
                plsc.addupdate_scatter(hist, [bkt], ones16)
                plsc.addupdate_scatter(coarse, [cbkt], ones16)
            return 0

        lax.fori_loop(0, NV // UNROLL, _hist_body, 0)

    for rr in range(ROWS_PER_W):
        row = wid * ROWS_PER_W + rr

        def _zero_hist(i, _):
            base = i * 256
            for u in range(16):
                hist[pl.ds(base + u * 16, 16)] = zeros16
            return 0

        lax.fori_loop(0, 256, _zero_hist, 0)

        def _zero_coarse(i, _):
            base = i * 256
            for u in range(16):
                coarse[pl.ds(base + u * 16, 16)] = zeros16
            return 0

        lax.fori_loop(0, 16, _zero_coarse, 0)

        start_chunk(row, 0, buf0, sem0)
        def _chunk_body(k, _):
            c0 = k * 2
            wait_chunk(row, c0, buf0, sem0)
            start_chunk(row, c0 + 1, buf1, sem1)
            process(buf0)
            wait_chunk(row, c0 + 1, buf1, sem1)

            @pl.when(k < NCHUNK // 2 - 1)
            def _():
                start_chunk(row, c0 + 2, buf0, sem0)

            process(buf1)
            return 0

        lax.fori_loop(0, NCHUNK // 2, _chunk_body, 0)

        def _cscan(g, carry):
            cum, gL, cumL, gU, cumU = carry
            vec = coarse[pl.ds(g * 16, 16)]
            tot = jnp.sum(vec)
            nxt = cum + tot
            condL = (cum <= T_LO) & (T_LO < nxt)
            condU = (cum <= T_UP) & (T_UP < nxt)
            gL = lax.select(condL, g, gL)
            cumL = lax.select(condL, cum, cumL)
            gU = lax.select(condU, g, gU)
            cumU = lax.select(condU, cum, cumU)
            return nxt, gL, cumL, gU, cumU

        z = jnp.int32(0)
        _, gL, cumL, gU, cumU = lax.fori_loop(0, 256, _cscan, (z, z, z, z, z))

        def _descend(ref, g, cumbef, t):
            vec = ref[pl.ds(g * 16, 16)]
            s = plsc.cumsum(vec)
            lane = jnp.sum((cumbef + s <= t).astype(jnp.int32))
            cnt = jnp.sum(jnp.where(iota == lane, vec, 0))
            cbef = cumbef + jnp.sum(jnp.where(iota == lane, s - vec, 0))
            return g * 16 + lane, cbef, cnt

        def _locate(g, cumbef, t):
            cb, cbef_c, _ = _descend(coarse, g, cumbef, t)
            fbin, cbef, cnt = _descend(hist, cb, cbef_c, t)
            klo = lax.shift_left(fbin, 16)
            khi = lax.shift_left(fbin + 1, 16)
            kv = jnp.where(iota == 0, klo, khi)
            bits = jnp.where(kv < 0,
                             lax.bitwise_xor(kv, _INT_MIN),
                             lax.bitwise_not(kv))
            fv = plsc.bitcast(bits, jnp.float32)
            neg_big = jnp.float32(-3.4e38)
            vlo = jnp.max(jnp.where(iota == 0, fv, neg_big))
            vhi = jnp.max(jnp.where(iota == 1, fv, neg_big))
            return vlo, vhi, cbef.astype(jnp.float32), cnt.astype(jnp.float32)

        vloL, vhiL, cbefL, cntL = _locate(gL, cumL, T_LO)
        vloU, vhiU, cbefU, cntU = _locate(gU, cumU, T_UP)
        fz = jnp.float32(0.0)
        resv = fz
        for lane_ix, val in ((0, vloL), (1, vhiL), (2, cbefL), (3, cntL),
                             (4, vloU), (5, vhiU), (6, cbefU), (7, cntU)):
            resv = jnp.where(iota == lane_ix, val, resv)
        outv[0, :] = resv
        pltpu.sync_copy(outv, out_hbm.at[row])


_sc_quantile = pl.kernel(
    _sc_body,
    out_type=jax.ShapeDtypeStruct((B, 1, 16), jnp.float32),
    mesh=plsc.VectorSubcoreMesh(core_axis_name="c", subcore_axis_name="s"),
    compiler_params=pltpu.CompilerParams(needs_layout_passes=False),
    scratch_types=[
        pltpu.VMEM((65536,), jnp.int32),
        pltpu.VMEM((4096,), jnp.int32),
        pltpu.VMEM((CROWS, 512), jnp.float32),
        pltpu.VMEM((CROWS, 512), jnp.float32),
        pltpu.VMEM((1, 16), jnp.float32),
        pltpu.SemaphoreType.DMA,
        pltpu.SemaphoreType.DMA,
    ],
)


def _norm_body(b_ref, x_ref, o_ref):
    def interp(vlo, vhi, cbef, cnt, pos):
        return vlo + (vhi - vlo) * ((jnp.float32(pos) - cbef
                                     + jnp.float32(0.5)) / cnt)

    lo = interp(b_ref[0, 0, 0], b_ref[0, 0, 1], b_ref[0, 0, 2],
                b_ref[0, 0, 3], POS_LO)
    up = interp(b_ref[0, 0, 4], b_ref[0, 0, 5], b_ref[0, 0, 6],
                b_ref[0, 0, 7], POS_UP)
    xv = x_ref[...]
    o_ref[...] = (jnp.maximum(jnp.minimum(xv, up), lo) - lo) / (up - lo)


_tc_normalize = pl.pallas_call(
    _norm_body,
    grid=(B,),
    in_specs=[
        pl.BlockSpec((1, 1, 16), lambda i: (i, 0, 0)),
        pl.BlockSpec((1, 1, 512, 512), lambda i: (i, 0, 0, 0)),
    ],
    out_specs=pl.BlockSpec((1, 1, 512, 512), lambda i: (i, 0, 0, 0)),
    out_shape=jax.ShapeDtypeStruct((B, 1, 512, 512), jnp.float32),
)


def kernel(x):
    xf = x.reshape(B, ROWS, 512)
    bounds = _sc_quantile(xf)
    return _tc_normalize(bounds, x)
